# baseline (speedup 1.0000x reference)
import dataclasses

import jax
import jax.numpy as jnp
from jax import lax
from jax.experimental import pallas as pl
from jax.experimental.pallas import tpu as pltpu
from jax.experimental.pallas import tpu_sc as plsc

PH_VOCAB, PH_DIM = 1000, 128
LANG_VOCAB, LANG_DIM = 1000, 16
SPK_VOCAB, SPK_DIM = 100000, 64
B, T = 1024, 200
SFX_DIM = LANG_DIM + SPK_DIM
OUT_DIM = PH_DIM + SFX_DIM
NC, NS = 2, 16
NW = NC * NS
BPW = B // NW
T0 = 128
T1 = T - T0


def _body(ph_tab, lang_tab, spk_tab, ph_ids, lang_ids, spk_ids, out,
          idx0, idx1, rows0, rows1, sfx0, sfx1, lang_rows, spk_rows,
          lid_v, sid_v,
          sem_i0, sem_i1, sem_g0, sem_g1, sem_o0, sem_o1, sem_r):
    c = lax.axis_index("c")
    s = lax.axis_index("s")
    wid = s * NC + c
    base = wid * BPW

    idxs = (idx0, idx1)
    rowss = (rows0, rows1)
    sfxs = (sfx0, sfx1)
    sems_i = (sem_i0, sem_i1)
    sems_g = (sem_g0, sem_g1)
    sems_o = (sem_o0, sem_o1)

    pltpu.async_copy(ph_ids.at[base], idx0, sem_i0)

    pltpu.sync_copy(lang_ids.at[pl.ds(base, BPW)], lid_v)
    pltpu.sync_copy(spk_ids.at[pl.ds(base, BPW)], sid_v)

    lanes = lax.broadcasted_iota(jnp.int32, (16,), 0)
    imin = jnp.int32(-2147483648)
    for ci in range(BPW // 16):
        lch = lid_v[pl.ds(ci * 16, 16)]
        sch = sid_v[pl.ds(ci * 16, 16)]
        for j in range(16):
            m = lanes == j
            lid = jnp.max(jnp.where(m, lch, imin))
            sid = jnp.max(jnp.where(m, sch, imin))
            i = ci * 16 + j
            pltpu.async_copy(lang_tab.at[lid], lang_rows.at[i], sem_r)
            pltpu.async_copy(spk_tab.at[sid], spk_rows.at[i], sem_r)
    for i in range(BPW):
        pltpu.make_async_copy(lang_tab.at[0], lang_rows.at[i], sem_r).wait()
        pltpu.make_async_copy(spk_tab.at[0], spk_rows.at[i], sem_r).wait()

    def half(i, p):
        b = base + i

        @pl.when(i >= 2)
        def _():
            pltpu.make_async_copy(
                rowss[p], out.at[b - 2, :, pl.ds(0, PH_DIM)], sems_o[p]).wait()
            pltpu.make_async_copy(
                sfxs[p], out.at[b - 2, :, pl.ds(PH_DIM, SFX_DIM)],
                sems_o[p]).wait()

        pltpu.make_async_copy(ph_ids.at[b], idxs[p], sems_i[p]).wait()
        pltpu.async_copy(ph_tab.at[idxs[p].at[pl.ds(0, T0)]],
                         rowss[p].at[pl.ds(0, T0)], sems_g[p])
        pltpu.async_copy(ph_tab.at[idxs[p].at[pl.ds(T0, T1)]],
                         rowss[p].at[pl.ds(T0, T1)], sems_g[p])

        @pl.when(i + 1 < BPW)
        def _():
            pltpu.async_copy(ph_ids.at[b + 1], idxs[1 - p], sems_i[1 - p])

        sfx = sfxs[p]
        lang_reg = lang_rows[i]
        spk_regs = [spk_rows[i, pl.ds(16 * j, 16)]
                    for j in range(SPK_DIM // 16)]

        @pl.loop(0, T)
        def _(t):
            sfx[t, pl.ds(0, 16)] = lang_reg
            for j, r in enumerate(spk_regs):
                sfx[t, pl.ds(LANG_DIM + 16 * j, 16)] = r

        pltpu.make_async_copy(ph_tab.at[idxs[p].at[pl.ds(0, T0)]],
                              rowss[p].at[pl.ds(0, T0)], sems_g[p]).wait()
        pltpu.make_async_copy(ph_tab.at[idxs[p].at[pl.ds(T0, T1)]],
                              rowss[p].at[pl.ds(T0, T1)], sems_g[p]).wait()
        pltpu.async_copy(rowss[p], out.at[b, :, pl.ds(0, PH_DIM)], sems_o[p])
        pltpu.async_copy(sfx, out.at[b, :, pl.ds(PH_DIM, SFX_DIM)], sems_o[p])

    @pl.loop(0, BPW, step=2)
    def _(i):
        half(i, 0)
        half(i + 1, 1)

    for p in (0, 1):
        b = base + BPW - 2 + p
        pltpu.make_async_copy(
            rowss[p], out.at[b, :, pl.ds(0, PH_DIM)], sems_o[p]).wait()
        pltpu.make_async_copy(
            sfxs[p], out.at[b, :, pl.ds(PH_DIM, SFX_DIM)], sems_o[p]).wait()


def kernel(phoneme_table, language_table, speaker_table, phoneme_ids,
           language_ids, speaker_ids):
    mesh = plsc.VectorSubcoreMesh(core_axis_name="c", subcore_axis_name="s")
    cp = pltpu.CompilerParams()
    if "needs_layout_passes" in pltpu.CompilerParams.__dataclass_fields__:
        cp = dataclasses.replace(cp, needs_layout_passes=False)
    if "use_tc_tiling_on_sc" in pltpu.CompilerParams.__dataclass_fields__:
        cp = dataclasses.replace(cp, use_tc_tiling_on_sc=True)
    f = pl.kernel(
        _body,
        out_type=jax.ShapeDtypeStruct((B, T, OUT_DIM), jnp.float32),
        mesh=mesh,
        compiler_params=cp,
        scratch_types=[
            pltpu.VMEM((T,), jnp.int32),
            pltpu.VMEM((T,), jnp.int32),
            pltpu.VMEM((T, PH_DIM), jnp.float32),
            pltpu.VMEM((T, PH_DIM), jnp.float32),
            pltpu.VMEM((T, SFX_DIM), jnp.float32),
            pltpu.VMEM((T, SFX_DIM), jnp.float32),
            pltpu.VMEM((BPW, LANG_DIM), jnp.float32),
            pltpu.VMEM((BPW, SPK_DIM), jnp.float32),
            pltpu.VMEM((BPW,), jnp.int32),
            pltpu.VMEM((BPW,), jnp.int32),
            pltpu.SemaphoreType.DMA,
            pltpu.SemaphoreType.DMA,
            pltpu.SemaphoreType.DMA,
            pltpu.SemaphoreType.DMA,
            pltpu.SemaphoreType.DMA,
            pltpu.SemaphoreType.DMA,
            pltpu.SemaphoreType.DMA,
        ],
    )
    return f(phoneme_table, language_table, speaker_table,
             phoneme_ids.astype(jnp.int32), language_ids.astype(jnp.int32),
             speaker_ids.astype(jnp.int32))


# speedup vs baseline: 1.3805x; 1.3805x over previous
import dataclasses

import jax
import jax.numpy as jnp
from jax import lax
from jax.experimental import pallas as pl
from jax.experimental.pallas import tpu as pltpu
from jax.experimental.pallas import tpu_sc as plsc

PH_VOCAB, PH_DIM = 1000, 128
LANG_VOCAB, LANG_DIM = 1000, 16
SPK_VOCAB, SPK_DIM = 100000, 64
B, T = 1024, 200
SFX_DIM = LANG_DIM + SPK_DIM
OUT_DIM = PH_DIM + SFX_DIM
NC, NS = 2, 16
NW = NC * NS
BPW = B // NW
T0 = 128
T1 = T - T0
T_BLK, B_BLK = 8, 512


def _sc_body(ph_tab, lang_tab, spk_tab, ph_ids, lang_ids, spk_ids,
             ph_g, sfx_hbm,
             idx0, idx1, rows0, rows1, sfx_rows, lid_v, sid_v,
             sem_i0, sem_i1, sem_g0, sem_g1, sem_o0, sem_o1, sem_r):
    c = lax.axis_index("c")
    s = lax.axis_index("s")
    wid = s * NC + c
    base = wid * BPW

    idxs = (idx0, idx1)
    rowss = (rows0, rows1)
    sems_i = (sem_i0, sem_i1)
    sems_g = (sem_g0, sem_g1)
    sems_o = (sem_o0, sem_o1)

    pltpu.async_copy(ph_ids.at[base], idx0, sem_i0)

    pltpu.sync_copy(lang_ids.at[pl.ds(base, BPW)], lid_v)
    pltpu.sync_copy(spk_ids.at[pl.ds(base, BPW)], sid_v)

    lanes = lax.broadcasted_iota(jnp.int32, (16,), 0)
    imin = jnp.int32(-2147483648)
    for ci in range(BPW // 16):
        lch = lid_v[pl.ds(ci * 16, 16)]
        sch = sid_v[pl.ds(ci * 16, 16)]
        for j in range(16):
            m = lanes == j
            lid = jnp.max(jnp.where(m, lch, imin))
            sid = jnp.max(jnp.where(m, sch, imin))
            i = ci * 16 + j
            pltpu.async_copy(lang_tab.at[lid],
                             sfx_rows.at[i, pl.ds(0, LANG_DIM)], sem_r)
            pltpu.async_copy(spk_tab.at[sid],
                             sfx_rows.at[i, pl.ds(LANG_DIM, SPK_DIM)], sem_r)
    for i in range(BPW):
        pltpu.make_async_copy(
            lang_tab.at[0], sfx_rows.at[i, pl.ds(0, LANG_DIM)], sem_r).wait()
        pltpu.make_async_copy(
            spk_tab.at[0], sfx_rows.at[i, pl.ds(LANG_DIM, SPK_DIM)],
            sem_r).wait()
    pltpu.async_copy(sfx_rows, sfx_hbm.at[pl.ds(base, BPW)], sem_r)

    def half(i, p):
        b = base + i

        @pl.when(i >= 2)
        def _():
            pltpu.make_async_copy(rowss[p], ph_g.at[b - 2], sems_o[p]).wait()

        pltpu.make_async_copy(ph_ids.at[b], idxs[p], sems_i[p]).wait()
        pltpu.async_copy(ph_tab.at[idxs[p].at[pl.ds(0, T0)]],
                         rowss[p].at[pl.ds(0, T0)], sems_g[p])
        pltpu.async_copy(ph_tab.at[idxs[p].at[pl.ds(T0, T1)]],
                         rowss[p].at[pl.ds(T0, T1)], sems_g[p])

        @pl.when(i + 1 < BPW)
        def _():
            pltpu.async_copy(ph_ids.at[b + 1], idxs[1 - p], sems_i[1 - p])

        pltpu.make_async_copy(ph_tab.at[idxs[p].at[pl.ds(0, T0)]],
                              rowss[p].at[pl.ds(0, T0)], sems_g[p]).wait()
        pltpu.make_async_copy(ph_tab.at[idxs[p].at[pl.ds(T0, T1)]],
                              rowss[p].at[pl.ds(T0, T1)], sems_g[p]).wait()
        pltpu.async_copy(rowss[p], ph_g.at[b], sems_o[p])

    @pl.loop(0, BPW, step=2)
    def _(i):
        half(i, 0)
        half(i + 1, 1)

    for p in (0, 1):
        pltpu.make_async_copy(
            rowss[p], ph_g.at[base + BPW - 2 + p], sems_o[p]).wait()
    pltpu.make_async_copy(sfx_rows, sfx_hbm.at[pl.ds(base, BPW)], sem_r).wait()


def _tc_body(ph_ref, sfx_ref, out_ref):
    sfx_t = jnp.swapaxes(sfx_ref[...], 0, 1)
    for t in range(T_BLK):
        ph_t = ph_ref[:, t, :]
        out_ref[t, 0:PH_DIM, :] = jnp.swapaxes(ph_t, 0, 1)
        out_ref[t, PH_DIM:OUT_DIM, :] = sfx_t[0:SFX_DIM, :]


def kernel(phoneme_table, language_table, speaker_table, phoneme_ids,
           language_ids, speaker_ids):
    mesh = plsc.VectorSubcoreMesh(core_axis_name="c", subcore_axis_name="s")
    cp = pltpu.CompilerParams()
    if "needs_layout_passes" in pltpu.CompilerParams.__dataclass_fields__:
        cp = dataclasses.replace(cp, needs_layout_passes=False)
    sc = pl.kernel(
        _sc_body,
        out_type=(jax.ShapeDtypeStruct((B, T, PH_DIM), jnp.float32),
                  jax.ShapeDtypeStruct((B, 128), jnp.float32)),
        mesh=mesh,
        compiler_params=cp,
        scratch_types=[
            pltpu.VMEM((T,), jnp.int32),
            pltpu.VMEM((T,), jnp.int32),
            pltpu.VMEM((T, PH_DIM), jnp.float32),
            pltpu.VMEM((T, PH_DIM), jnp.float32),
            pltpu.VMEM((BPW, 128), jnp.float32),
            pltpu.VMEM((BPW,), jnp.int32),
            pltpu.VMEM((BPW,), jnp.int32),
            pltpu.SemaphoreType.DMA,
            pltpu.SemaphoreType.DMA,
            pltpu.SemaphoreType.DMA,
            pltpu.SemaphoreType.DMA,
            pltpu.SemaphoreType.DMA,
            pltpu.SemaphoreType.DMA,
            pltpu.SemaphoreType.DMA,
        ],
    )
    ph_g, sfx = sc(phoneme_table, language_table, speaker_table,
                   phoneme_ids.astype(jnp.int32),
                   language_ids.astype(jnp.int32),
                   speaker_ids.astype(jnp.int32))

    p = pl.pallas_call(
        _tc_body,
        grid=(T // T_BLK, B // B_BLK),
        in_specs=[
            pl.BlockSpec((B_BLK, T_BLK, PH_DIM), lambda i, j: (j, i, 0)),
            pl.BlockSpec((B_BLK, 128), lambda i, j: (j, 0)),
        ],
        out_specs=pl.BlockSpec((T_BLK, OUT_DIM, B_BLK), lambda i, j: (i, 0, j)),
        out_shape=jax.ShapeDtypeStruct((T, OUT_DIM, B), jnp.float32),
    )(ph_g, sfx)
    return jnp.transpose(p, (2, 0, 1))
